# baseline (speedup 1.0000x reference)
import functools

import jax
import jax.numpy as jnp
from jax import lax
from jax.experimental import pallas as pl
from jax.experimental.pallas import tpu as pltpu
from jax.experimental.pallas import tpu_sc as plsc

B = 16
FD = 24
N = FD * FD * FD
TOPK = 60
KPAD = 64
NMS_TOPK = 20
THRESHOLD = 0.15
NMS_TH = 0.05
STRIDE = 4.0
REG = 36
NCH = 3 * REG
SCHUNKS = NCH * KPAD // 128
SPAD = 64
ROWS0 = 32
ROWS1 = SCHUNKS - ROWS0


def _topk_tc(cls_ref, sv_ref, si_ref, gs_ref, go_ref):
    x = cls_ref[...]
    s = jax.nn.sigmoid(x)
    lane_n = lax.broadcasted_iota(jnp.int32, (B, N), 1)
    vals, idxs = [], []
    for _ in range(TOPK):
        m = jnp.max(s, axis=1, keepdims=True)
        i = jnp.min(jnp.where(s == m, lane_n, jnp.int32(N)),
                    axis=1, keepdims=True)
        vals.append(m)
        idxs.append(i)
        s = jnp.where(lane_n == i, -jnp.inf, s)
    sv = jnp.concatenate(vals + [jnp.zeros((B, KPAD - TOPK), jnp.float32)], axis=1)
    si = jnp.concatenate(idxs + [jnp.zeros((B, KPAD - TOPK), jnp.int32)], axis=1)
    sv_ref[...] = sv
    si_ref[...] = si

    n2 = jnp.concatenate([si, si], axis=1)
    n3 = jnp.broadcast_to(n2[:, None, :], (B, SPAD, 128))
    b3 = lax.broadcasted_iota(jnp.int32, (B, SPAD, 128), 0)
    j3 = lax.broadcasted_iota(jnp.int32, (B, SPAD, 128), 1)
    t3 = lax.broadcasted_iota(jnp.int32, (B, SPAD, 128), 2)
    c3 = j3 * 2 + (t3 >= KPAD).astype(jnp.int32)
    gs_ref[...] = jnp.where(c3 < NCH, (b3 * NCH + c3) * N + n3, 0)

    no = jnp.broadcast_to(n2[:, None, :], (B, 8, 128))
    bo = lax.broadcasted_iota(jnp.int32, (B, 8, 128), 0)
    jo = lax.broadcasted_iota(jnp.int32, (B, 8, 128), 1)
    to = lax.broadcasted_iota(jnp.int32, (B, 8, 128), 2)
    co = jo * 2 + (to >= KPAD).astype(jnp.int32)
    go_ref[...] = jnp.where(co < 3, (bo * 3 + jnp.minimum(co, 2)) * N + no, 0)


@functools.cache
def _make_gather_sc():
    @functools.partial(
        pl.kernel,
        mesh=plsc.VectorSubcoreMesh(core_axis_name="c", subcore_axis_name="s"),
        out_type=[jax.ShapeDtypeStruct((B, NCH * KPAD), jnp.float32),
                  jax.ShapeDtypeStruct((B, 256), jnp.float32)],
        scratch_types=[pltpu.VMEM((ROWS0, 128), jnp.int32),
                       pltpu.VMEM((8, 128), jnp.int32),
                       pltpu.VMEM((ROWS0 * 128,), jnp.float32),
                       pltpu.VMEM((256,), jnp.float32),
                       pltpu.SemaphoreType.DMA],
    )
    def gather_sc(gs_hbm, go_hbm, shp_hbm, off_hbm, dist_out, offs_out,
                  idx_v, idxo_v, buf_v, bufo_v, sem):
        wid = lax.axis_index("s") * 2 + lax.axis_index("c")
        b = wid // 2
        h = wid % 2
        pltpu.sync_copy(gs_hbm.at[b, pl.ds(h * ROWS0, ROWS0), :], idx_v)

        @pl.when(h == 0)
        def _():
            cps = [pltpu.async_copy(shp_hbm.at[idx_v.at[j]],
                                    buf_v.at[pl.ds(j * 128, 128)], sem)
                   for j in range(ROWS0)]
            for cp in cps:
                cp.wait()
            pltpu.sync_copy(buf_v, dist_out.at[b, pl.ds(0, ROWS0 * 128)])

        @pl.when(h == 1)
        def _():
            cps = [pltpu.async_copy(shp_hbm.at[idx_v.at[j]],
                                    buf_v.at[pl.ds(j * 128, 128)], sem)
                   for j in range(ROWS1)]
            for cp in cps:
                cp.wait()
            pltpu.sync_copy(buf_v.at[pl.ds(0, ROWS1 * 128)],
                            dist_out.at[b, pl.ds(ROWS0 * 128, ROWS1 * 128)])
            pltpu.sync_copy(go_hbm.at[b], idxo_v)
            c0 = pltpu.async_copy(off_hbm.at[idxo_v.at[0]],
                                  bufo_v.at[pl.ds(0, 128)], sem)
            c1 = pltpu.async_copy(off_hbm.at[idxo_v.at[1]],
                                  bufo_v.at[pl.ds(128, 128)], sem)
            c0.wait()
            c1.wait()
            pltpu.sync_copy(bufo_v, offs_out.at[b])

    return gather_sc


def _gather_sc(gs, go, shp_flat, off_flat):
    return _make_gather_sc()(gs, go, shp_flat, off_flat)


def _post_tc(sv_ref, si_ref, dist_ref, offs_ref, out_ref):
    out_ref[...] = jnp.full((B, TOPK, 8), -1.0, jnp.float32)
    dist = dist_ref[...]
    wreg = lax.broadcasted_iota(jnp.int32, (B, REG, KPAD), 1).astype(jnp.float32)
    shp = []
    for a in range(3):
        v = dist[:, a * REG:(a + 1) * REG, :]
        m = jnp.max(v, axis=1, keepdims=True)
        e = jnp.exp(v - m)
        shp.append(jnp.sum(e * wreg, axis=1) / jnp.sum(e, axis=1))

    n = si_ref[...]
    zf = (n // (FD * FD)).astype(jnp.float32)
    yf = ((n // FD) % FD).astype(jnp.float32)
    xf = (n % FD).astype(jnp.float32)
    offs = offs_ref[...]
    cz = (zf + offs[:, 0, :]) * STRIDE
    cy = (yf + offs[:, 1, :]) * STRIDE
    cx = (xf + offs[:, 2, :]) * STRIDE
    dz = shp[0] * STRIDE
    dy = shp[1] * STRIDE
    dx = shp[2] * STRIDE

    lane = lax.broadcasted_iota(jnp.int32, (B, KPAD), 1)
    sval = sv_ref[...]
    s = jnp.where((lane < TOPK) & (sval > THRESHOLD), sval, -jnp.inf)

    loz, hiz = cz - dz * 0.5, cz + dz * 0.5
    loy, hiy = cy - dy * 0.5, cy + dy * 0.5
    lox, hix = cx - dx * 0.5, cx + dx * 0.5
    vol = dz * dy * dx

    def sel(a, one):
        return jnp.sum(jnp.where(one, a, 0.0), axis=1, keepdims=True)

    for k in range(NMS_TOPK):
        m = jnp.max(s, axis=1, keepdims=True)
        ok = m > -jnp.inf
        i = jnp.min(jnp.where(s == m, lane, jnp.int32(KPAD)),
                    axis=1, keepdims=True)
        one = lane == i
        bz, by, bx = sel(cz, one), sel(cy, one), sel(cx, one)
        bd, bh, bw = sel(dz, one), sel(dy, one), sel(dx, one)
        iz = jnp.maximum(jnp.minimum(hiz, bz + bd * 0.5)
                         - jnp.maximum(loz, bz - bd * 0.5), 0.0)
        iy = jnp.maximum(jnp.minimum(hiy, by + bh * 0.5)
                         - jnp.maximum(loy, by - bh * 0.5), 0.0)
        ix = jnp.maximum(jnp.minimum(hix, bx + bw * 0.5)
                         - jnp.maximum(lox, bx - bw * 0.5), 0.0)
        inter = iz * iy * ix
        iou = inter / (bd * bh * bw + vol - inter + 1e-8)
        s = jnp.where(ok & ((iou > NMS_TH) | one), -jnp.inf, s)
        neg = jnp.full_like(m, -1.0)
        row = jnp.concatenate(
            [jnp.where(ok, 1.0, neg), jnp.where(ok, m, neg),
             jnp.where(ok, bz, neg), jnp.where(ok, by, neg),
             jnp.where(ok, bx, neg), jnp.where(ok, bd, neg),
             jnp.where(ok, bh, neg), jnp.where(ok, bw, neg)], axis=1)
        out_ref[:, k, :] = row


def kernel(Cls, Shape, Offset):
    cls2 = Cls.reshape(B, N)
    sv, si, gs, go = pl.pallas_call(
        _topk_tc,
        out_shape=[jax.ShapeDtypeStruct((B, KPAD), jnp.float32),
                   jax.ShapeDtypeStruct((B, KPAD), jnp.int32),
                   jax.ShapeDtypeStruct((B, SPAD, 128), jnp.int32),
                   jax.ShapeDtypeStruct((B, 8, 128), jnp.int32)],
    )(cls2)
    dist_flat, offs_flat = _gather_sc(gs, go, Shape.reshape(-1), Offset.reshape(-1))
    return pl.pallas_call(
        _post_tc,
        out_shape=jax.ShapeDtypeStruct((B, TOPK, 8), jnp.float32),
    )(sv, si, dist_flat.reshape(B, NCH, KPAD), offs_flat.reshape(B, 4, KPAD))


# speedup vs baseline: 7.8979x; 7.8979x over previous
import jax
import jax.numpy as jnp
from jax import lax
from jax.experimental import pallas as pl
from jax.experimental.pallas import tpu as pltpu

B = 16
FD = 24
N = FD * FD * FD
TOPK = 60
KPAD = 64
NMS_TOPK = 20
THRESHOLD = 0.15
NMS_TH = 0.05
STRIDE = 4.0
REG = 36
NCH = 3 * REG
VROWS = B * N
OBROWS = 3 * FD * FD
SLABS = TOPK * 8


def _topk_tc(cls_ref, sv_ref, si_ref):
    x = cls_ref[...]
    s = jax.nn.sigmoid(x)
    lane_n = lax.broadcasted_iota(jnp.int32, (B, N), 1)
    vals, idxs = [], []
    for _ in range(TOPK):
        m = jnp.max(s, axis=1, keepdims=True)
        i = jnp.min(jnp.where(s == m, lane_n, jnp.int32(N)),
                    axis=1, keepdims=True)
        vals.append(m)
        idxs.append(i)
        s = jnp.where(lane_n == i, -jnp.inf, s)
    sv_ref[...] = jnp.concatenate(
        vals + [jnp.zeros((B, KPAD - TOPK), jnp.float32)], axis=1)
    si_ref[...] = jnp.concatenate(
        idxs + [jnp.zeros((B, KPAD - TOPK), jnp.int32)], axis=1)


def _gather_tc(si_s, si_ref, offb_ref, shp_ref, dist_ref, osel_ref,
               slab_v, sem):
    b = pl.program_id(0)
    cps = []
    for k in range(TOPK):
        row = b * N + si_s[b, k]
        t8 = pl.multiple_of((row // 8) * 8, 8)
        cp = pltpu.make_async_copy(shp_ref.at[pl.ds(t8, 8), :],
                                   slab_v.at[pl.ds(k * 8, 8), :], sem)
        cp.start()
        cps.append(cp)

    n = si_ref[0]
    offb = offb_ref[0]
    rvec = (n // (FD * FD)) * FD + (n // FD) % FD
    xv = n % FD
    xio = lax.broadcasted_iota(jnp.int32, (FD, KPAD), 0)
    xhot = (xio == xv).astype(jnp.float32)
    qio = lax.broadcasted_iota(jnp.int32, (OBROWS, KPAD), 0)
    for c in range(3):
        rhot = (qio == rvec + c * (FD * FD)).astype(jnp.float32)
        oc = lax.dot_general(offb, rhot, (((0,), (0,)), ((), ())),
                             precision=lax.Precision.HIGHEST,
                             preferred_element_type=jnp.float32)
        osel_ref[0, c, :] = jnp.sum(oc * xhot, axis=0)

    for cp in cps:
        cp.wait()
    rm8 = (b * N + n) % 8
    kio = lax.broadcasted_iota(jnp.int32, (SLABS, KPAD), 1)
    qio2 = lax.broadcasted_iota(jnp.int32, (SLABS, KPAD), 0)
    sel = (qio2 == kio * 8 + rm8).astype(jnp.float32)
    sel = jnp.where(lax.broadcasted_iota(jnp.int32, (SLABS, KPAD), 1) < TOPK,
                    sel, 0.0)
    dist_ref[0] = lax.dot_general(slab_v[...], sel, (((0,), (0,)), ((), ())),
                                  precision=lax.Precision.HIGHEST,
                                  preferred_element_type=jnp.float32)


def _post_tc(sv_ref, si_ref, dist_ref, osel_ref, out_ref):
    out_ref[...] = jnp.full((B, TOPK, 8), -1.0, jnp.float32)
    dist = dist_ref[...]
    wreg = lax.broadcasted_iota(jnp.int32, (B, REG, KPAD), 1).astype(jnp.float32)
    shp = []
    for a in range(3):
        v = dist[:, a * REG:(a + 1) * REG, :]
        m = jnp.max(v, axis=1, keepdims=True)
        e = jnp.exp(v - m)
        shp.append(jnp.sum(e * wreg, axis=1) / jnp.sum(e, axis=1))

    n = si_ref[...]
    zf = (n // (FD * FD)).astype(jnp.float32)
    yf = ((n // FD) % FD).astype(jnp.float32)
    xf = (n % FD).astype(jnp.float32)
    osel = osel_ref[...]
    cz = (zf + osel[:, 0, :]) * STRIDE
    cy = (yf + osel[:, 1, :]) * STRIDE
    cx = (xf + osel[:, 2, :]) * STRIDE
    dz = shp[0] * STRIDE
    dy = shp[1] * STRIDE
    dx = shp[2] * STRIDE

    lane = lax.broadcasted_iota(jnp.int32, (B, KPAD), 1)
    sval = sv_ref[...]
    s = jnp.where((lane < TOPK) & (sval > THRESHOLD), sval, -jnp.inf)

    loz, hiz = cz - dz * 0.5, cz + dz * 0.5
    loy, hiy = cy - dy * 0.5, cy + dy * 0.5
    lox, hix = cx - dx * 0.5, cx + dx * 0.5
    vol = dz * dy * dx

    def sel(a, one):
        return jnp.sum(jnp.where(one, a, 0.0), axis=1, keepdims=True)

    for k in range(NMS_TOPK):
        m = jnp.max(s, axis=1, keepdims=True)
        ok = m > -jnp.inf
        i = jnp.min(jnp.where(s == m, lane, jnp.int32(KPAD)),
                    axis=1, keepdims=True)
        one = lane == i
        bz, by, bx = sel(cz, one), sel(cy, one), sel(cx, one)
        bd, bh, bw = sel(dz, one), sel(dy, one), sel(dx, one)
        iz = jnp.maximum(jnp.minimum(hiz, bz + bd * 0.5)
                         - jnp.maximum(loz, bz - bd * 0.5), 0.0)
        iy = jnp.maximum(jnp.minimum(hiy, by + bh * 0.5)
                         - jnp.maximum(loy, by - bh * 0.5), 0.0)
        ix = jnp.maximum(jnp.minimum(hix, bx + bw * 0.5)
                         - jnp.maximum(lox, bx - bw * 0.5), 0.0)
        inter = iz * iy * ix
        iou = inter / (bd * bh * bw + vol - inter + 1e-8)
        s = jnp.where(ok & ((iou > NMS_TH) | one), -jnp.inf, s)
        neg = jnp.full_like(m, -1.0)
        row = jnp.concatenate(
            [jnp.where(ok, 1.0, neg), jnp.where(ok, m, neg),
             jnp.where(ok, bz, neg), jnp.where(ok, by, neg),
             jnp.where(ok, bx, neg), jnp.where(ok, bd, neg),
             jnp.where(ok, bh, neg), jnp.where(ok, bw, neg)], axis=1)
        out_ref[:, k, :] = row


def _run_gather(si, shp_t, offb):
    si3 = si.reshape(B, 1, KPAD)
    return pl.pallas_call(
        _gather_tc,
        grid_spec=pltpu.PrefetchScalarGridSpec(
            num_scalar_prefetch=1,
            grid=(B,),
            in_specs=[
                pl.BlockSpec((1, 1, KPAD), lambda b, s: (b, 0, 0)),
                pl.BlockSpec((1, OBROWS, FD), lambda b, s: (b, 0, 0)),
                pl.BlockSpec(memory_space=pl.ANY),
            ],
            out_specs=[
                pl.BlockSpec((1, NCH, KPAD), lambda b, s: (b, 0, 0)),
                pl.BlockSpec((1, 3, KPAD), lambda b, s: (b, 0, 0)),
            ],
            scratch_shapes=[
                pltpu.VMEM((SLABS, NCH), jnp.float32),
                pltpu.SemaphoreType.DMA,
            ],
        ),
        out_shape=[jax.ShapeDtypeStruct((B, NCH, KPAD), jnp.float32),
                   jax.ShapeDtypeStruct((B, 3, KPAD), jnp.float32)],
    )(si, si3, offb, shp_t)


def kernel(Cls, Shape, Offset):
    cls2 = Cls.reshape(B, N)
    shp_t = Shape.transpose(0, 2, 3, 4, 1).reshape(VROWS, NCH)
    offb = Offset.reshape(B, OBROWS, FD)
    sv, si = pl.pallas_call(
        _topk_tc,
        out_shape=[jax.ShapeDtypeStruct((B, KPAD), jnp.float32),
                   jax.ShapeDtypeStruct((B, KPAD), jnp.int32)],
    )(cls2)
    dist, osel = _run_gather(si, shp_t, offb)
    return pl.pallas_call(
        _post_tc,
        out_shape=jax.ShapeDtypeStruct((B, TOPK, 8), jnp.float32),
    )(sv, si, dist, osel)


# speedup vs baseline: 8.1950x; 1.0376x over previous
import jax
import jax.numpy as jnp
from jax import lax
from jax.experimental import pallas as pl
from jax.experimental.pallas import tpu as pltpu

B = 16
FD = 24
N = FD * FD * FD
TOPK = 60
KPAD = 64
NMS_TOPK = 20
THRESHOLD = 0.15
NMS_TH = 0.05
STRIDE = 4.0
REG = 36
NCH = 3 * REG
VROWS = B * N
OBROWS = 3 * FD * FD
SLABS = TOPK * 8


def _topk_tc(cls_ref, sv_ref, si_ref):
    x = cls_ref[...]
    s = jax.nn.sigmoid(x)
    lane_n = lax.broadcasted_iota(jnp.int32, (B, N), 1)
    vals, idxs = [], []
    for _ in range(TOPK):
        m = jnp.max(s, axis=1, keepdims=True)
        i = jnp.min(jnp.where(s == m, lane_n, jnp.int32(N)),
                    axis=1, keepdims=True)
        vals.append(m)
        idxs.append(i)
        s = jnp.where(lane_n == i, -jnp.inf, s)
    sv_ref[...] = jnp.concatenate(
        vals + [jnp.zeros((B, KPAD - TOPK), jnp.float32)], axis=1)
    si_ref[...] = jnp.concatenate(
        idxs + [jnp.zeros((B, KPAD - TOPK), jnp.int32)], axis=1)


def _gather_tc(si_s, si_ref, offb_ref, shp_ref, dist_ref, osel_ref, sem):
    b = pl.program_id(0)
    cps = []
    for k in range(TOPK):
        row = b * N + si_s[b, k]
        cp = pltpu.make_async_copy(shp_ref.at[pl.ds(row, 1), :],
                                   dist_ref.at[0, pl.ds(k, 1), :], sem)
        cp.start()
        cps.append(cp)

    lane64 = lax.broadcasted_iota(jnp.int32, (1, KPAD), 1)
    xio = lax.broadcasted_iota(jnp.int32, (FD,), 0)
    accs = [jnp.zeros((1, KPAD), jnp.float32) for _ in range(3)]
    for k in range(TOPK):
        nk = si_s[b, k]
        r = (nk // (FD * FD)) * FD + (nk // FD) % FD
        xhot = xio == nk % FD
        khot = lane64 == k
        for c in range(3):
            rowv = offb_ref[0, r + c * (FD * FD), :]
            val = jnp.sum(jnp.where(xhot, rowv, 0.0))
            accs[c] = jnp.where(khot, val, accs[c])
    for c in range(3):
        osel_ref[0, c, :] = accs[c][0]

    for cp in cps:
        cp.wait()


def _post_tc(sv_ref, si_ref, dist_ref, osel_ref, out_ref):
    out_ref[...] = jnp.full((B, TOPK, 8), -1.0, jnp.float32)
    dist = dist_ref[...]
    wreg = lax.broadcasted_iota(jnp.int32, (B, KPAD, REG), 2).astype(jnp.float32)
    shp = []
    for a in range(3):
        v = dist[:, :, a * REG:(a + 1) * REG]
        m = jnp.max(v, axis=2, keepdims=True)
        e = jnp.exp(v - m)
        shp.append(jnp.sum(e * wreg, axis=2) / jnp.sum(e, axis=2))

    n = si_ref[...]
    zf = (n // (FD * FD)).astype(jnp.float32)
    yf = ((n // FD) % FD).astype(jnp.float32)
    xf = (n % FD).astype(jnp.float32)
    osel = osel_ref[...]
    cz = (zf + osel[:, 0, :]) * STRIDE
    cy = (yf + osel[:, 1, :]) * STRIDE
    cx = (xf + osel[:, 2, :]) * STRIDE
    dz = shp[0] * STRIDE
    dy = shp[1] * STRIDE
    dx = shp[2] * STRIDE

    lane = lax.broadcasted_iota(jnp.int32, (B, KPAD), 1)
    sval = sv_ref[...]
    s = jnp.where((lane < TOPK) & (sval > THRESHOLD), sval, -jnp.inf)

    loz, hiz = cz - dz * 0.5, cz + dz * 0.5
    loy, hiy = cy - dy * 0.5, cy + dy * 0.5
    lox, hix = cx - dx * 0.5, cx + dx * 0.5
    vol = dz * dy * dx

    def sel(a, one):
        return jnp.sum(jnp.where(one, a, 0.0), axis=1, keepdims=True)

    for k in range(NMS_TOPK):
        m = jnp.max(s, axis=1, keepdims=True)
        ok = m > -jnp.inf
        i = jnp.min(jnp.where(s == m, lane, jnp.int32(KPAD)),
                    axis=1, keepdims=True)
        one = lane == i
        bz, by, bx = sel(cz, one), sel(cy, one), sel(cx, one)
        bd, bh, bw = sel(dz, one), sel(dy, one), sel(dx, one)
        iz = jnp.maximum(jnp.minimum(hiz, bz + bd * 0.5)
                         - jnp.maximum(loz, bz - bd * 0.5), 0.0)
        iy = jnp.maximum(jnp.minimum(hiy, by + bh * 0.5)
                         - jnp.maximum(loy, by - bh * 0.5), 0.0)
        ix = jnp.maximum(jnp.minimum(hix, bx + bw * 0.5)
                         - jnp.maximum(lox, bx - bw * 0.5), 0.0)
        inter = iz * iy * ix
        iou = inter / (bd * bh * bw + vol - inter + 1e-8)
        s = jnp.where(ok & ((iou > NMS_TH) | one), -jnp.inf, s)
        neg = jnp.full_like(m, -1.0)
        row = jnp.concatenate(
            [jnp.where(ok, 1.0, neg), jnp.where(ok, m, neg),
             jnp.where(ok, bz, neg), jnp.where(ok, by, neg),
             jnp.where(ok, bx, neg), jnp.where(ok, bd, neg),
             jnp.where(ok, bh, neg), jnp.where(ok, bw, neg)], axis=1)
        out_ref[:, k, :] = row


def _run_gather(si, shp_t, offb):
    si3 = si.reshape(B, 1, KPAD)
    return pl.pallas_call(
        _gather_tc,
        grid_spec=pltpu.PrefetchScalarGridSpec(
            num_scalar_prefetch=1,
            grid=(B,),
            in_specs=[
                pl.BlockSpec((1, 1, KPAD), lambda b, s: (b, 0, 0)),
                pl.BlockSpec((1, OBROWS, FD), lambda b, s: (b, 0, 0)),
                pl.BlockSpec(memory_space=pl.ANY),
            ],
            out_specs=[
                pl.BlockSpec((1, KPAD, NCH), lambda b, s: (b, 0, 0)),
                pl.BlockSpec((1, 3, KPAD), lambda b, s: (b, 0, 0)),
            ],
            scratch_shapes=[
                pltpu.SemaphoreType.DMA,
            ],
        ),
        out_shape=[jax.ShapeDtypeStruct((B, KPAD, NCH), jnp.float32),
                   jax.ShapeDtypeStruct((B, 3, KPAD), jnp.float32)],
    )(si, si3, offb, shp_t)


def kernel(Cls, Shape, Offset):
    cls2 = Cls.reshape(B, N)
    shp_t = Shape.transpose(0, 2, 3, 4, 1).reshape(VROWS, NCH)
    offb = Offset.reshape(B, OBROWS, FD)
    sv, si = pl.pallas_call(
        _topk_tc,
        out_shape=[jax.ShapeDtypeStruct((B, KPAD), jnp.float32),
                   jax.ShapeDtypeStruct((B, KPAD), jnp.int32)],
    )(cls2)
    dist, osel = _run_gather(si, shp_t, offb)
    return pl.pallas_call(
        _post_tc,
        out_shape=jax.ShapeDtypeStruct((B, TOPK, 8), jnp.float32),
    )(sv, si, dist, osel)
